# baseline (speedup 1.0000x reference)
import functools

import jax
import jax.numpy as jnp
from jax.experimental import pallas as pl
from jax.experimental.pallas import tpu as pltpu


def _round_up(x, m):
    return ((x + m - 1) // m) * m


def _conv_stack_kernel(x_ref, w_hbm, b_ref, bcol_ref, eye_ref, o_ref,
                       act_a, act_b, w_ref, w_sems, *, n_layers, ksize,
                       seq_len, pad_lo, front):
    bt = x_ref.shape[0]
    cp = x_ref.shape[1]
    m = bt * seq_len
    base = front - pad_lo
    halo_len = act_a.shape[1]
    tail = front + seq_len
    first_step = pl.program_id(0) == 0

    @pl.when(first_step)
    def _start_weight_copies():
        for l in range(n_layers):
            pltpu.make_async_copy(
                w_hbm.at[l], w_ref.at[l], w_sems.at[l]).start()

    for buf in (act_a, act_b):
        buf[:, pl.ds(0, front), :] = jnp.zeros((bt, front, cp), buf.dtype)
        if halo_len > tail:
            buf[:, pl.ds(tail, halo_len - tail), :] = jnp.zeros(
                (bt, halo_len - tail, cp), buf.dtype)

    xcat = jnp.concatenate([x_ref[i] for i in range(bt)], axis=1)
    s0 = jax.lax.dot_general(
        xcat, eye_ref[...], (((0,), (0,)), ((), ())),
        preferred_element_type=jnp.float32)
    act_a[:, pl.ds(front, seq_len), :] = s0.reshape(bt, seq_len, cp)

    bufs = (act_a, act_b)
    for layer in range(n_layers):
        src = bufs[layer % 2]
        dst = bufs[(layer + 1) % 2]
        last = layer == n_layers - 1

        @pl.when(first_step)
        def _wait_weights(layer=layer):
            pltpu.make_async_copy(
                w_hbm.at[layer], w_ref.at[layer], w_sems.at[layer]).wait()

        acc = None
        for k in range(ksize):
            lhs = src[:, pl.ds(base + k, seq_len), :].reshape(m, cp)
            if last:
                d = jax.lax.dot_general(
                    w_ref[layer, k], lhs, (((0,), (1,)), ((), ())),
                    preferred_element_type=jnp.float32)
            else:
                d = jnp.dot(lhs, w_ref[layer, k],
                            preferred_element_type=jnp.float32)
            acc = d if acc is None else acc + d

        if last:
            y = acc + bcol_ref[layer].astype(jnp.float32)
            for i in range(bt):
                o_ref[i] = y[:, i * seq_len:(i + 1) * seq_len]
        else:
            y = acc + b_ref[layer].astype(jnp.float32)
            dst[:, pl.ds(front, seq_len), :] = (
                y.reshape(bt, seq_len, cp).astype(dst.dtype))


def kernel(x, w_padded, b_padded):
    B, C, L = x.shape
    n_layers, K, _, Cp = w_padded.shape
    pad_lo = (K - 1) // 2
    pad_hi = K - 1 - pad_lo
    front = _round_up(max(pad_lo, 1), 8)
    bt = min(B, max(1, 1024 // max(L, 1)))
    Bp = _round_up(B, bt)
    if Bp != B:
        x = jnp.pad(x, ((0, Bp - B), (0, 0), (0, 0)))
    grid = (Bp // bt,)
    halo_len = front + L + pad_hi

    b_col = jnp.swapaxes(b_padded, 1, 2)
    eye = jnp.eye(Cp, dtype=x.dtype)

    fn = functools.partial(
        _conv_stack_kernel, n_layers=n_layers, ksize=K,
        seq_len=L, pad_lo=pad_lo, front=front)
    out = pl.pallas_call(
        fn,
        out_shape=jax.ShapeDtypeStruct((Bp, Cp, L), x.dtype),
        grid_spec=pltpu.PrefetchScalarGridSpec(
            num_scalar_prefetch=0,
            grid=grid,
            in_specs=[
                pl.BlockSpec((bt, Cp, L), lambda i: (i, 0, 0)),
                pl.BlockSpec(memory_space=pl.ANY),
                pl.BlockSpec((n_layers, 1, Cp), lambda i: (0, 0, 0)),
                pl.BlockSpec((n_layers, Cp, 1), lambda i: (0, 0, 0)),
                pl.BlockSpec((Cp, Cp), lambda i: (0, 0)),
            ],
            out_specs=pl.BlockSpec((bt, Cp, L), lambda i: (i, 0, 0)),
            scratch_shapes=[
                pltpu.VMEM((bt, halo_len, Cp), x.dtype),
                pltpu.VMEM((bt, halo_len, Cp), x.dtype),
                pltpu.VMEM((n_layers, K, Cp, Cp), w_padded.dtype),
                pltpu.SemaphoreType.DMA((n_layers,)),
            ],
        ),
        compiler_params=pltpu.CompilerParams(
            dimension_semantics=("parallel",),
            vmem_limit_bytes=56 * 1024 * 1024,
        ),
    )(x, w_padded, b_padded, b_col, eye)
    return out[:B]


# speedup vs baseline: 1.0342x; 1.0342x over previous
import functools

import jax
import jax.numpy as jnp
from jax.experimental import pallas as pl
from jax.experimental.pallas import tpu as pltpu


def _round_up(x, m):
    return ((x + m - 1) // m) * m


def _conv_stack_kernel(x_ref, w_ref, b_ref, bcol_ref, eye_ref, o_ref,
                       act_a, act_b, *, n_layers, ksize,
                       seq_len, pad_lo, front):
    bt = x_ref.shape[0]
    cp = x_ref.shape[1]
    m = bt * seq_len
    base = front - pad_lo
    halo_len = act_a.shape[1]
    tail = front + seq_len
    @pl.when(pl.program_id(0) == 0)
    def _zero_halos():
        for buf in (act_a, act_b):
            buf[:, pl.ds(0, front), :] = jnp.zeros(
                (bt, front, cp), buf.dtype)
            if halo_len > tail:
                buf[:, pl.ds(tail, halo_len - tail), :] = jnp.zeros(
                    (bt, halo_len - tail, cp), buf.dtype)

    xcat = jnp.concatenate([x_ref[i] for i in range(bt)], axis=1)
    s0 = jax.lax.dot_general(
        xcat, eye_ref[...], (((0,), (0,)), ((), ())),
        preferred_element_type=jnp.float32)
    act_a[:, pl.ds(front, seq_len), :] = s0.reshape(bt, seq_len, cp)

    bufs = (act_a, act_b)
    for layer in range(n_layers):
        src = bufs[layer % 2]
        dst = bufs[(layer + 1) % 2]
        last = layer == n_layers - 1

        acc = None
        for k in range(ksize):
            lhs = src[:, pl.ds(base + k, seq_len), :].reshape(m, cp)
            if last:
                d = jax.lax.dot_general(
                    w_ref[layer, k], lhs, (((0,), (1,)), ((), ())),
                    preferred_element_type=jnp.float32)
            else:
                d = jnp.dot(lhs, w_ref[layer, k],
                            preferred_element_type=jnp.float32)
            acc = d if acc is None else acc + d

        if last:
            y = acc + bcol_ref[layer].astype(jnp.float32)
            for i in range(bt):
                o_ref[i] = y[:, i * seq_len:(i + 1) * seq_len]
        else:
            y = acc + b_ref[layer].astype(jnp.float32)
            dst[:, pl.ds(front, seq_len), :] = (
                y.reshape(bt, seq_len, cp).astype(dst.dtype))


def kernel(x, w_padded, b_padded):
    B, C, L = x.shape
    n_layers, K, _, Cp = w_padded.shape
    pad_lo = (K - 1) // 2
    pad_hi = K - 1 - pad_lo
    front = _round_up(max(pad_lo, 1), 8)
    bt = min(B, max(1, 1024 // max(L, 1)))
    Bp = _round_up(B, bt)
    if Bp != B:
        x = jnp.pad(x, ((0, Bp - B), (0, 0), (0, 0)))
    grid = (Bp // bt,)
    halo_len = front + L + pad_hi

    b_col = jnp.swapaxes(b_padded, 1, 2)
    eye = jnp.eye(Cp, dtype=x.dtype)

    fn = functools.partial(
        _conv_stack_kernel, n_layers=n_layers, ksize=K,
        seq_len=L, pad_lo=pad_lo, front=front)
    out = pl.pallas_call(
        fn,
        out_shape=jax.ShapeDtypeStruct((Bp, Cp, L), x.dtype),
        grid_spec=pltpu.PrefetchScalarGridSpec(
            num_scalar_prefetch=0,
            grid=grid,
            in_specs=[
                pl.BlockSpec((bt, Cp, L), lambda i: (i, 0, 0)),
                pl.BlockSpec((n_layers, K, Cp, Cp), lambda i: (0, 0, 0, 0)),
                pl.BlockSpec((n_layers, 1, Cp), lambda i: (0, 0, 0)),
                pl.BlockSpec((n_layers, Cp, 1), lambda i: (0, 0, 0)),
                pl.BlockSpec((Cp, Cp), lambda i: (0, 0)),
            ],
            out_specs=pl.BlockSpec((bt, Cp, L), lambda i: (i, 0, 0)),
            scratch_shapes=[
                pltpu.VMEM((bt, halo_len, Cp), x.dtype),
                pltpu.VMEM((bt, halo_len, Cp), x.dtype),
            ],
        ),
        compiler_params=pltpu.CompilerParams(
            dimension_semantics=("parallel",),
            vmem_limit_bytes=56 * 1024 * 1024,
        ),
    )(x, w_padded, b_padded, b_col, eye)
    return out[:B]


# speedup vs baseline: 1.0514x; 1.0166x over previous
import functools

import jax
import jax.numpy as jnp
from jax.experimental import pallas as pl
from jax.experimental.pallas import tpu as pltpu


def _round_up(x, m):
    return ((x + m - 1) // m) * m


def _conv_stack_kernel(x_ref, w_ref, b_ref, bcol_ref, o_ref,
                       act_a, act_b, eye_ref, *, n_layers, ksize,
                       seq_len, pad_lo, front):
    bt = x_ref.shape[0]
    cp = x_ref.shape[1]
    m = bt * seq_len
    base = front - pad_lo
    halo_len = act_a.shape[1]
    tail = front + seq_len
    @pl.when(pl.program_id(0) == 0)
    def _setup_scratch():
        for buf in (act_a, act_b):
            buf[:, pl.ds(0, front), :] = jnp.zeros(
                (bt, front, cp), buf.dtype)
            if halo_len > tail:
                buf[:, pl.ds(tail, halo_len - tail), :] = jnp.zeros(
                    (bt, halo_len - tail, cp), buf.dtype)
        rows = jax.lax.broadcasted_iota(jnp.int32, (cp, cp), 0)
        cols = jax.lax.broadcasted_iota(jnp.int32, (cp, cp), 1)
        eye_ref[...] = jnp.where(rows == cols,
                                 jnp.float32(1), jnp.float32(0))

    xcat = jnp.concatenate([x_ref[i] for i in range(bt)], axis=1)
    s0 = jax.lax.dot_general(
        xcat, eye_ref[...], (((0,), (0,)), ((), ())),
        preferred_element_type=jnp.float32)
    act_a[:, pl.ds(front, seq_len), :] = s0.reshape(bt, seq_len, cp)

    bufs = (act_a, act_b)
    for layer in range(n_layers):
        src = bufs[layer % 2]
        dst = bufs[(layer + 1) % 2]
        last = layer == n_layers - 1

        acc = None
        for k in range(ksize):
            lhs = src[:, pl.ds(base + k, seq_len), :].reshape(m, cp)
            if last:
                d = jax.lax.dot_general(
                    w_ref[layer, k], lhs, (((0,), (1,)), ((), ())),
                    preferred_element_type=jnp.float32)
            else:
                d = jnp.dot(lhs, w_ref[layer, k],
                            preferred_element_type=jnp.float32)
            acc = d if acc is None else acc + d

        if last:
            y = acc + bcol_ref[layer].astype(jnp.float32)
            for i in range(bt):
                o_ref[i] = y[:, i * seq_len:(i + 1) * seq_len]
        else:
            y = acc + b_ref[layer].astype(jnp.float32)
            dst[:, pl.ds(front, seq_len), :] = (
                y.reshape(bt, seq_len, cp).astype(dst.dtype))


def kernel(x, w_padded, b_padded):
    B, C, L = x.shape
    n_layers, K, _, Cp = w_padded.shape
    pad_lo = (K - 1) // 2
    pad_hi = K - 1 - pad_lo
    front = _round_up(max(pad_lo, 1), 8)
    bt = min(B, max(1, 1024 // max(L, 1)))
    Bp = _round_up(B, bt)
    if Bp != B:
        x = jnp.pad(x, ((0, Bp - B), (0, 0), (0, 0)))
    grid = (Bp // bt,)
    halo_len = front + L + pad_hi

    b_col = jnp.swapaxes(b_padded, 1, 2)

    fn = functools.partial(
        _conv_stack_kernel, n_layers=n_layers, ksize=K,
        seq_len=L, pad_lo=pad_lo, front=front)
    out = pl.pallas_call(
        fn,
        out_shape=jax.ShapeDtypeStruct((Bp, Cp, L), x.dtype),
        grid_spec=pltpu.PrefetchScalarGridSpec(
            num_scalar_prefetch=0,
            grid=grid,
            in_specs=[
                pl.BlockSpec((bt, Cp, L), lambda i: (i, 0, 0)),
                pl.BlockSpec((n_layers, K, Cp, Cp), lambda i: (0, 0, 0, 0)),
                pl.BlockSpec((n_layers, 1, Cp), lambda i: (0, 0, 0)),
                pl.BlockSpec((n_layers, Cp, 1), lambda i: (0, 0, 0)),
            ],
            out_specs=pl.BlockSpec((bt, Cp, L), lambda i: (i, 0, 0)),
            scratch_shapes=[
                pltpu.VMEM((bt, halo_len, Cp), x.dtype),
                pltpu.VMEM((bt, halo_len, Cp), x.dtype),
                pltpu.VMEM((Cp, Cp), x.dtype),
            ],
        ),
        compiler_params=pltpu.CompilerParams(
            dimension_semantics=("parallel",),
            vmem_limit_bytes=56 * 1024 * 1024,
        ),
    )(x, w_padded, b_padded, b_col)
    return out[:B]


# speedup vs baseline: 1.0736x; 1.0211x over previous
import functools

import jax
import jax.numpy as jnp
from jax.experimental import pallas as pl
from jax.experimental.pallas import tpu as pltpu


def _round_up(x, m):
    return ((x + m - 1) // m) * m


def _conv_stack_kernel(x_ref, w_ref, b_ref, o_ref,
                       act_a, act_b, eye_ref, bcol_ref, *, n_layers, ksize,
                       seq_len, pad_lo, front):
    bt = x_ref.shape[0]
    cp = x_ref.shape[1]
    m = bt * seq_len
    base = front - pad_lo
    halo_len = act_a.shape[1]
    tail = front + seq_len
    @pl.when(pl.program_id(0) == 0)
    def _setup_scratch():
        for buf in (act_a, act_b):
            buf[:, pl.ds(0, front), :] = jnp.zeros(
                (bt, front, cp), buf.dtype)
            if halo_len > tail:
                buf[:, pl.ds(tail, halo_len - tail), :] = jnp.zeros(
                    (bt, halo_len - tail, cp), buf.dtype)
        rows = jax.lax.broadcasted_iota(jnp.int32, (cp, cp), 0)
        cols = jax.lax.broadcasted_iota(jnp.int32, (cp, cp), 1)
        eye_ref[...] = jnp.where(rows == cols,
                                 jnp.float32(1), jnp.float32(0))
        bcol_ref[...] = jax.lax.dot_general(
            eye_ref[...], b_ref[n_layers - 1], (((1,), (1,)), ((), ())),
            preferred_element_type=jnp.float32)

    xcat = jnp.concatenate([x_ref[i] for i in range(bt)], axis=1)
    s0 = jax.lax.dot_general(
        xcat, eye_ref[...], (((0,), (0,)), ((), ())),
        preferred_element_type=jnp.float32)
    act_a[:, pl.ds(front, seq_len), :] = s0.reshape(bt, seq_len, cp)

    bufs = (act_a, act_b)
    for layer in range(n_layers):
        src = bufs[layer % 2]
        dst = bufs[(layer + 1) % 2]
        last = layer == n_layers - 1

        acc = None
        for k in range(ksize):
            lhs = src[:, pl.ds(base + k, seq_len), :].reshape(m, cp)
            if last:
                d = jax.lax.dot_general(
                    w_ref[layer, k], lhs, (((0,), (1,)), ((), ())),
                    preferred_element_type=jnp.float32)
            else:
                d = jnp.dot(lhs, w_ref[layer, k],
                            preferred_element_type=jnp.float32)
            acc = d if acc is None else acc + d

        if last:
            y = acc + bcol_ref[...]
            for i in range(bt):
                o_ref[i] = y[:, i * seq_len:(i + 1) * seq_len]
        else:
            y = acc + b_ref[layer].astype(jnp.float32)
            dst[:, pl.ds(front, seq_len), :] = (
                y.reshape(bt, seq_len, cp).astype(dst.dtype))


def kernel(x, w_padded, b_padded):
    B, C, L = x.shape
    n_layers, K, _, Cp = w_padded.shape
    pad_lo = (K - 1) // 2
    pad_hi = K - 1 - pad_lo
    front = _round_up(max(pad_lo, 1), 8)
    bt = min(B, max(1, 1024 // max(L, 1)))
    Bp = _round_up(B, bt)
    if Bp != B:
        x = jnp.pad(x, ((0, Bp - B), (0, 0), (0, 0)))
    grid = (Bp // bt,)
    halo_len = front + L + pad_hi

    fn = functools.partial(
        _conv_stack_kernel, n_layers=n_layers, ksize=K,
        seq_len=L, pad_lo=pad_lo, front=front)
    out = pl.pallas_call(
        fn,
        out_shape=jax.ShapeDtypeStruct((Bp, Cp, L), x.dtype),
        grid_spec=pltpu.PrefetchScalarGridSpec(
            num_scalar_prefetch=0,
            grid=grid,
            in_specs=[
                pl.BlockSpec((bt, Cp, L), lambda i: (i, 0, 0)),
                pl.BlockSpec((n_layers, K, Cp, Cp), lambda i: (0, 0, 0, 0)),
                pl.BlockSpec((n_layers, 1, Cp), lambda i: (0, 0, 0)),
            ],
            out_specs=pl.BlockSpec((bt, Cp, L), lambda i: (i, 0, 0)),
            scratch_shapes=[
                pltpu.VMEM((bt, halo_len, Cp), x.dtype),
                pltpu.VMEM((bt, halo_len, Cp), x.dtype),
                pltpu.VMEM((Cp, Cp), x.dtype),
                pltpu.VMEM((Cp, 1), jnp.float32),
            ],
        ),
        compiler_params=pltpu.CompilerParams(
            dimension_semantics=("parallel",),
            vmem_limit_bytes=56 * 1024 * 1024,
        ),
    )(x, w_padded, b_padded)
    return out[:B]
